# Initial kernel scaffold; baseline (speedup 1.0000x reference)
import jax
import jax.numpy as jnp
from jax.experimental import pallas as pl


def kernel(indices, embed_table, fc_weight, fc_bias):
    raise NotImplementedError("write your pallas kernel here")


# speedup vs baseline: 11.7583x; 11.7583x over previous
import functools

import jax
import jax.numpy as jnp
from jax import lax
from jax.experimental import pallas as pl
from jax.experimental.pallas import tpu as pltpu
from jax.experimental.pallas import tpu_sc as plsc

VOCAB = 20000
EMBED = 128
OUT = 6
BATCH = 4096
SEQ = 200
LANES = 16
NWORK = 32
BPW = BATCH // NWORK
PROJ_BLK = 2000


def _proj_body(e_ref, w_ref, b_ref, o_ref):
    o_ref[...] = lax.dot_general(
        e_ref[...], w_ref[...],
        (((1,), (1,)), ((), ())),
        preferred_element_type=jnp.float32,
    ) + b_ref[...]


def _project(embed_table, wp, bp):
    return pl.pallas_call(
        _proj_body,
        grid=(VOCAB // PROJ_BLK,),
        in_specs=[
            pl.BlockSpec((PROJ_BLK, EMBED), lambda i: (i, 0)),
            pl.BlockSpec((LANES, EMBED), lambda i: (0, 0)),
            pl.BlockSpec((1, LANES), lambda i: (0, 0)),
        ],
        out_specs=pl.BlockSpec((PROJ_BLK, LANES), lambda i: (i, 0)),
        out_shape=jax.ShapeDtypeStruct((VOCAB, LANES), jnp.float32),
    )(embed_table, wp, bp)


def _make_sc_pool():
    mesh = plsc.VectorSubcoreMesh(core_axis_name="c", subcore_axis_name="s")

    @functools.partial(
        pl.kernel,
        out_type=jax.ShapeDtypeStruct((BATCH, LANES), jnp.float32),
        mesh=mesh,
        compiler_params=pltpu.CompilerParams(use_tc_tiling_on_sc=False),
        scratch_types=[
            pltpu.VMEM((SEQ,), jnp.int32),
            pltpu.VMEM((SEQ, LANES), jnp.float32),
            pltpu.VMEM((BPW, LANES), jnp.float32),
            pltpu.SemaphoreType.DMA,
        ],
    )
    def pool(p_hbm, idx_hbm, out_hbm, idx_v, rows_v, ost_v, sem):
        wid = lax.axis_index("c") * 16 + lax.axis_index("s")
        base = wid * BPW

        def batch_body(j, carry):
            b = base + j
            pltpu.sync_copy(idx_hbm.at[pl.ds(b * SEQ, SEQ)], idx_v)
            pltpu.async_copy(p_hbm.at[idx_v], rows_v, sem).wait()

            def red(i, accs):
                i8 = i * 8
                return tuple(accs[t] + rows_v[i8 + t] for t in range(8))

            accs = lax.fori_loop(
                0, SEQ // 8, red,
                tuple(jnp.zeros((LANES,), jnp.float32) for _ in range(8)))
            acc = (((accs[0] + accs[1]) + (accs[2] + accs[3]))
                   + ((accs[4] + accs[5]) + (accs[6] + accs[7])))
            ost_v[j] = acc * (1.0 / SEQ)
            return carry

        lax.fori_loop(0, BPW, batch_body, 0)
        pltpu.sync_copy(ost_v, out_hbm.at[pl.ds(base, BPW)])

    return pool


_sc_pool = _make_sc_pool()


def kernel(indices, embed_table, fc_weight, fc_bias):
    wp = jnp.zeros((LANES, EMBED), jnp.float32).at[:OUT].set(fc_weight)
    bp = jnp.zeros((1, LANES), jnp.float32).at[0, :OUT].set(fc_bias)
    p = _project(embed_table, wp, bp)
    out16 = _sc_pool(p, indices.reshape(-1))
    return out16[:, :OUT][:, None, :]


# speedup vs baseline: 30.5365x; 2.5970x over previous
import functools

import jax
import jax.numpy as jnp
from jax import lax
from jax.experimental import pallas as pl
from jax.experimental.pallas import tpu as pltpu
from jax.experimental.pallas import tpu_sc as plsc

VOCAB = 20000
EMBED = 128
OUT = 6
BATCH = 4096
SEQ = 200
LANES = 16
NWORK = 32
BPW = BATCH // NWORK
PROJ_BLK = 2000


def _proj_body(e_ref, w_ref, b_ref, o_ref):
    o_ref[...] = lax.dot_general(
        e_ref[...], w_ref[...],
        (((1,), (1,)), ((), ())),
        preferred_element_type=jnp.float32,
    ) + b_ref[...]


def _project(embed_table, wp, bp):
    return pl.pallas_call(
        _proj_body,
        grid=(VOCAB // PROJ_BLK,),
        in_specs=[
            pl.BlockSpec((PROJ_BLK, EMBED), lambda i: (i, 0)),
            pl.BlockSpec((LANES, EMBED), lambda i: (0, 0)),
            pl.BlockSpec((1, LANES), lambda i: (0, 0)),
        ],
        out_specs=pl.BlockSpec((PROJ_BLK, LANES), lambda i: (i, 0)),
        out_shape=jax.ShapeDtypeStruct((VOCAB, LANES), jnp.float32),
    )(embed_table, wp, bp)


CB = 8
NCH = BPW // CB
CHROWS = CB * SEQ


def _make_sc_pool():
    mesh = plsc.VectorSubcoreMesh(core_axis_name="c", subcore_axis_name="s")

    @functools.partial(
        pl.kernel,
        out_type=jax.ShapeDtypeStruct((BATCH, LANES), jnp.float32),
        mesh=mesh,
        compiler_params=pltpu.CompilerParams(use_tc_tiling_on_sc=False),
        scratch_types=[
            pltpu.VMEM((BPW * SEQ,), jnp.int32),
            pltpu.VMEM((2, CHROWS, LANES), jnp.float32),
            pltpu.VMEM((BPW, LANES), jnp.float32),
            pltpu.SemaphoreType.DMA,
            pltpu.SemaphoreType.DMA,
        ],
    )
    def pool(p_hbm, idx_hbm, out_hbm, idx_v, rows_v, ost_v, sem0, sem1):
        wid = lax.axis_index("c") * 16 + lax.axis_index("s")
        base = wid * BPW
        pltpu.sync_copy(idx_hbm.at[pl.ds(base * SEQ, BPW * SEQ)], idx_v)
        sems = (sem0, sem1)

        def issue(c, p):
            pltpu.async_copy(
                p_hbm.at[idx_v.at[pl.ds(c * CHROWS, CHROWS)]],
                rows_v.at[p], sems[p])

        def wait(p):
            pltpu.make_async_copy(
                p_hbm.at[idx_v.at[pl.ds(0, CHROWS)]],
                rows_v.at[p], sems[p]).wait()

        def reduce_chunk(c, p):
            for k in range(CB):
                def red(i, accs):
                    r0 = k * SEQ + i * 8
                    return tuple(accs[t] + rows_v[p, r0 + t] for t in range(8))

                accs = lax.fori_loop(
                    0, SEQ // 8, red,
                    tuple(jnp.zeros((LANES,), jnp.float32) for _ in range(8)))
                acc = (((accs[0] + accs[1]) + (accs[2] + accs[3]))
                       + ((accs[4] + accs[5]) + (accs[6] + accs[7])))
                ost_v[c * CB + k] = acc * (1.0 / SEQ)

        issue(0, 0)
        issue(1, 1)

        def chunk_pair(h, carry):
            for parity in range(2):
                c = h * 2 + parity
                wait(parity)
                reduce_chunk(c, parity)
                issue(c + 2, parity)
            return carry

        lax.fori_loop(0, NCH // 2 - 1, chunk_pair, 0)
        for parity in range(2):
            wait(parity)
            reduce_chunk(NCH - 2 + parity, parity)

        pltpu.sync_copy(ost_v, out_hbm.at[pl.ds(base, BPW)])

    return pool


_sc_pool = _make_sc_pool()


def kernel(indices, embed_table, fc_weight, fc_bias):
    wp = jnp.zeros((LANES, EMBED), jnp.float32).at[:OUT].set(fc_weight)
    bp = jnp.zeros((1, LANES), jnp.float32).at[0, :OUT].set(fc_bias)
    p = _project(embed_table, wp, bp)
    out16 = _sc_pool(p, indices.reshape(-1))
    return out16[:, :OUT][:, None, :]
